# Initial kernel scaffold; baseline (speedup 1.0000x reference)
import jax
import jax.numpy as jnp
from jax.experimental import pallas as pl


def kernel(x, edge_index_0, edge_index_1, edge_index_2, W0, att_src0, att_dst0, bias0, W1, att_src1, att_dst1, bias1, W2, att_src2, att_dst2, bias2, att_w, att_b):
    raise NotImplementedError("write your pallas kernel here")


# speedup vs baseline: 42.9204x; 42.9204x over previous
import functools

import jax
import jax.numpy as jnp
from jax import lax
from jax.experimental import pallas as pl
from jax.experimental.pallas import tpu as pltpu
from jax.experimental.pallas import tpu_sc as plsc

N = 10000
E = 320000
D = 128
H = 8
C = D // H
NE = 3

NC = 2
NS = 16
NW = NC * NS
CHUNK = 128
EP = 323584
EDGES_PER_TILE = EP // NW
CHUNKS_PER_TILE = EDGES_PER_TILE // CHUNK
WF = D + 2 * H
ROWS = 10112
RPT = ROWS // NS

BLK = 400
GRID = N // BLK


def _prep_body(x_ref, w0, w1, w2, s0, s1, s2, d0, d1, d2,
               ha0, ha1, ha2, ad0, ad1, ad2):
    xb = x_ref[...]
    ws = (w0, w1, w2)
    ss = (s0, s1, s2)
    ds = (d0, d1, d2)
    ha_o = (ha0, ha1, ha2)
    ad_o = (ad0, ad1, ad2)
    z8 = jnp.zeros((xb.shape[0], H), jnp.float32)
    for t in range(NE):
        h = jnp.dot(xb, ws[t][...], preferred_element_type=jnp.float32)
        a_s = jnp.dot(h, ss[t][...], preferred_element_type=jnp.float32)
        a_d = jnp.dot(h, ds[t][...], preferred_element_type=jnp.float32)
        ha_o[t][...] = jnp.concatenate([h, a_s, z8], axis=1)
        ad_o[t][...] = jnp.concatenate([a_d, z8], axis=1)


_prep = pl.pallas_call(
    _prep_body,
    grid=(GRID,),
    in_specs=[pl.BlockSpec((BLK, D), lambda i: (i, 0))]
    + [pl.BlockSpec((D, D), lambda i: (0, 0))] * 3
    + [pl.BlockSpec((D, H), lambda i: (0, 0))] * 6,
    out_specs=[pl.BlockSpec((BLK, WF), lambda i: (i, 0))] * 3
    + [pl.BlockSpec((BLK, 2 * H), lambda i: (i, 0))] * 3,
    out_shape=[jax.ShapeDtypeStruct((N, WF), jnp.float32)] * 3
    + [jax.ShapeDtypeStruct((N, 2 * H), jnp.float32)] * 3,
)


def _sc_body(ha0, ha1, ha2, adt0, adt1, adt2,
             s0, d0, s1, d1, s2, d2, zrows,
             out, srcv, dstv, ha_buf, ad_buf, acc, sem_a, sem_b):
    cid = lax.axis_index("c")
    sid = lax.axis_index("s")
    has = (ha0, ha1, ha2)
    ads = (adt0, adt1, adt2)
    srcs = (s0, s1, s2)
    dsts = (d0, d1, d2)
    base_tile = cid * (EP // NC) + sid * EDGES_PER_TILE

    for t in range(NE):
        pltpu.sync_copy(zrows, acc.at[pl.ds(sid * RPT, RPT)])
        plsc.subcore_barrier()

        def chunk_body(c, _, t=t):
            base = base_tile + c * CHUNK
            pltpu.sync_copy(srcs[t].at[pl.ds(base, CHUNK)], srcv)
            pltpu.sync_copy(dsts[t].at[pl.ds(base, CHUNK)], dstv)
            ga = pltpu.async_copy(has[t].at[srcv], ha_buf, sem_a)
            gb = pltpu.async_copy(ads[t].at[dstv], ad_buf, sem_b)
            ga.wait()
            gb.wait()

            def edge_body(k, _):
                asv = ha_buf[k, pl.ds(D, 2 * H)]
                adv = ad_buf[k, pl.ds(0, 2 * H)]
                e = asv + adv
                e = jnp.where(e > 0, e, e * jnp.float32(0.2))
                w = jnp.exp(e)
                ha_buf[k, pl.ds(D, 2 * H)] = w
                kk = jnp.full((16,), k, jnp.int32)
                for j in range(H):
                    wj = plsc.load_gather(
                        ha_buf, [kk, jnp.full((16,), D + j, jnp.int32)])
                    hv = ha_buf[k, pl.ds(j * 16, 16)]
                    ha_buf[k, pl.ds(j * 16, 16)] = hv * wj
                return 0

            lax.fori_loop(0, CHUNK, edge_body, 0)
            pltpu.sync_copy(ha_buf, acc.at[dstv], add=True)
            return 0

        lax.fori_loop(0, CHUNKS_PER_TILE, chunk_body, 0)
        plsc.subcore_barrier()
        pltpu.sync_copy(acc.at[pl.ds(sid * RPT, RPT)],
                        out.at[t, cid, pl.ds(sid * RPT, RPT)])
        plsc.subcore_barrier()


_sc_edge = pl.kernel(
    _sc_body,
    out_type=jax.ShapeDtypeStruct((NE, NC, ROWS, WF), jnp.float32),
    mesh=plsc.VectorSubcoreMesh(core_axis_name="c", subcore_axis_name="s"),
    scratch_types=[
        pltpu.VMEM((CHUNK,), jnp.int32),
        pltpu.VMEM((CHUNK,), jnp.int32),
        pltpu.VMEM((CHUNK, WF), jnp.float32),
        pltpu.VMEM((CHUNK, 2 * H), jnp.float32),
        pltpu.VMEM_SHARED((ROWS, WF), jnp.float32),
        pltpu.SemaphoreType.DMA,
        pltpu.SemaphoreType.DMA,
    ],
    compiler_params=pltpu.CompilerParams(use_tc_tiling_on_sc=False,
                                         needs_layout_passes=False),
)


def _combine_body(acc_ref, ha0, ha1, ha2, ad0, ad1, ad2,
                  b0, b1, b2, exp8_ref, attw_ref, attb_ref, out_ref):
    exp8 = exp8_ref[...]
    attw = attw_ref[...]
    accv = acc_ref[...]
    has = (ha0, ha1, ha2)
    adt = (ad0, ad1, ad2)
    bs = (b0, b1, b2)
    res = jnp.zeros((BLK, D), jnp.float32)
    for t in range(NE):
        m = accv[t, 0, :, :D] + accv[t, 1, :, :D]
        den = accv[t, 0, :, D:D + H] + accv[t, 1, :, D:D + H]
        hav = has[t][...]
        h = hav[:, :D]
        a_s = hav[:, D:D + H]
        a_d = adt[t][...][:, :H]
        e = a_s + a_d
        e = jnp.where(e > 0, e, e * jnp.float32(0.2))
        wself = jnp.exp(e)
        den_t = den + wself
        wx = jnp.dot(wself, exp8, preferred_element_type=jnp.float32)
        m_t = m + wx * h
        dfull = jnp.dot(den_t, exp8, preferred_element_type=jnp.float32)
        o = m_t / (dfull + jnp.float32(1e-16)) + bs[t][...]
        att = jnp.sum(o * attw, axis=1, keepdims=True) + attb_ref[...]
        res = res + att * o
    out_ref[...] = res


_combine = pl.pallas_call(
    _combine_body,
    grid=(GRID,),
    in_specs=[pl.BlockSpec((NE, NC, BLK, WF), lambda i: (0, 0, i, 0))]
    + [pl.BlockSpec((BLK, WF), lambda i: (i, 0))] * 3
    + [pl.BlockSpec((BLK, 2 * H), lambda i: (i, 0))] * 3
    + [pl.BlockSpec((1, D), lambda i: (0, 0))] * 3
    + [pl.BlockSpec((H, D), lambda i: (0, 0)),
       pl.BlockSpec((1, D), lambda i: (0, 0)),
       pl.BlockSpec((1, 1), lambda i: (0, 0))],
    out_specs=pl.BlockSpec((BLK, D), lambda i: (i, 0)),
    out_shape=jax.ShapeDtypeStruct((N, D), jnp.float32),
)


def kernel(x, edge_index_0, edge_index_1, edge_index_2,
           W0, att_src0, att_dst0, bias0,
           W1, att_src1, att_dst1, bias1,
           W2, att_src2, att_dst2, bias2,
           att_w, att_b):
    f32 = jnp.float32
    onehot = (jnp.arange(D, dtype=jnp.int32)[:, None] // C
              == jnp.arange(H, dtype=jnp.int32)[None, :]).astype(f32)
    a_srcs = [onehot * a.reshape(D)[:, None]
              for a in (att_src0, att_src1, att_src2)]
    a_dsts = [onehot * a.reshape(D)[:, None]
              for a in (att_dst0, att_dst1, att_dst2)]

    ha0, ha1, ha2, adt0, adt1, adt2 = _prep(x, W0, W1, W2, *a_srcs, *a_dsts)

    pad = EP - E
    srcs, dsts = [], []
    for ei in (edge_index_0, edge_index_1, edge_index_2):
        srcs.append(jnp.concatenate([ei[0], jnp.zeros((pad,), jnp.int32)]))
        dsts.append(jnp.concatenate([ei[1], jnp.full((pad,), N, jnp.int32)]))
    zrows = jnp.zeros((RPT, WF), f32)

    acc = _sc_edge(ha0, ha1, ha2, adt0, adt1, adt2,
                   srcs[0], dsts[0], srcs[1], dsts[1], srcs[2], dsts[2],
                   zrows)

    res = _combine(acc, ha0, ha1, ha2, adt0, adt1, adt2,
                   bias0.reshape(1, D), bias1.reshape(1, D),
                   bias2.reshape(1, D), onehot.T,
                   att_w.reshape(1, D), att_b.reshape(1, 1))
    return res


# speedup vs baseline: 93.2473x; 2.1726x over previous
import functools

import jax
import jax.numpy as jnp
from jax import lax
from jax.experimental import pallas as pl
from jax.experimental.pallas import tpu as pltpu
from jax.experimental.pallas import tpu_sc as plsc

N = 10000
E = 320000
D = 128
H = 8
C = D // H
NE = 3

NC = 2
NS = 16
NW = NC * NS
CHUNK = 112
CPT = 90
NBUF = 2
EDGES_PER_TILE = CPT * CHUNK
EP = NW * EDGES_PER_TILE
WF = D + 2 * H
ROWS = 10016
RPT = ROWS // NS

BLK = 400
GRID = N // BLK


def _prep_body(x_ref, w0, w1, w2, s0, s1, s2, d0, d1, d2,
               ha0, ha1, ha2, ad0, ad1, ad2):
    xb = x_ref[...]
    ws = (w0, w1, w2)
    ss = (s0, s1, s2)
    ds = (d0, d1, d2)
    ha_o = (ha0, ha1, ha2)
    ad_o = (ad0, ad1, ad2)
    z8 = jnp.zeros((xb.shape[0], H), jnp.float32)
    for t in range(NE):
        h = jnp.dot(xb, ws[t][...], preferred_element_type=jnp.float32)
        a_s = jnp.dot(h, ss[t][...], preferred_element_type=jnp.float32)
        a_d = jnp.dot(h, ds[t][...], preferred_element_type=jnp.float32)
        ha_o[t][...] = jnp.concatenate([h, a_s, z8], axis=1)
        ad_o[t][...] = jnp.concatenate([a_d, z8], axis=1)


_prep = pl.pallas_call(
    _prep_body,
    grid=(GRID,),
    in_specs=[pl.BlockSpec((BLK, D), lambda i: (i, 0))]
    + [pl.BlockSpec((D, D), lambda i: (0, 0))] * 3
    + [pl.BlockSpec((D, H), lambda i: (0, 0))] * 6,
    out_specs=[pl.BlockSpec((BLK, WF), lambda i: (i, 0))] * 3
    + [pl.BlockSpec((BLK, 2 * H), lambda i: (i, 0))] * 3,
    out_shape=[jax.ShapeDtypeStruct((N, WF), jnp.float32)] * 3
    + [jax.ShapeDtypeStruct((N, 2 * H), jnp.float32)] * 3,
)


def _sc_body(ha0, ha1, ha2, adt0, adt1, adt2,
             ids0, ids1, ids2, zrows,
             out, idxb, hb0, hb1, ab0, ab1,
             acc, semh0, semh1, sema0, sema1):
    cid = lax.axis_index("c")
    sid = lax.axis_index("s")
    has = (ha0, ha1, ha2)
    ads = (adt0, adt1, adt2)
    ids = (ids0, ids1, ids2)
    hbufs = (hb0, hb1)
    abufs = (ab0, ab1)
    semh = (semh0, semh1)
    sema = (sema0, sema1)
    tb = (cid * NS + sid) * CPT

    for t in range(NE):
        pltpu.sync_copy(zrows, acc.at[pl.ds(sid * RPT, RPT)])
        plsc.subcore_barrier()

        for b in range(NBUF):
            pltpu.sync_copy(ids[t].at[tb + b], idxb.at[b])
            pltpu.async_copy(has[t].at[idxb.at[b, 0]], hbufs[b], semh[b])
            pltpu.async_copy(ads[t].at[idxb.at[b, 1]], abufs[b], sema[b])

        def outer(g, _, t=t):
            for b in range(NBUF):
                c = g * NBUF + b
                hb = hbufs[b]
                ab = abufs[b]
                pltpu.make_async_copy(has[t].at[idxb.at[b, 0]],
                                      hb, semh[b]).wait()
                pltpu.make_async_copy(ads[t].at[idxb.at[b, 1]],
                                      ab, sema[b]).wait()

                def edge_body(k, _, hb=hb, ab=ab):
                    asv = hb[k, pl.ds(D, 2 * H)]
                    adv = ab[k, pl.ds(0, 2 * H)]
                    e = asv + adv
                    e = jnp.where(e > 0, e, e * jnp.float32(0.2))
                    w = jnp.exp(e)
                    hb[k, pl.ds(D, 2 * H)] = w
                    for j in range(H):
                        wj = jnp.full((16,), w[j], jnp.float32)
                        hv = hb[k, pl.ds(j * 16, 16)]
                        hb[k, pl.ds(j * 16, 16)] = hv * wj
                    return 0

                lax.fori_loop(0, CHUNK, edge_body, 0, unroll=2)
                pltpu.sync_copy(hb, acc.at[idxb.at[b, 1]], add=True)
                nxt = c + NBUF

                @pl.when(nxt < CPT)
                def _issue(t=t, b=b, hb=hb, ab=ab, nxt=nxt):
                    pltpu.sync_copy(ids[t].at[tb + nxt], idxb.at[b])
                    pltpu.async_copy(has[t].at[idxb.at[b, 0]],
                                     hb, semh[b])
                    pltpu.async_copy(ads[t].at[idxb.at[b, 1]],
                                     ab, sema[b])
            return 0

        lax.fori_loop(0, CPT // NBUF, outer, 0)
        plsc.subcore_barrier()
        pltpu.sync_copy(acc.at[pl.ds(sid * RPT, RPT)],
                        out.at[t, cid, pl.ds(sid * RPT, RPT)])
        plsc.subcore_barrier()


_sc_edge = pl.kernel(
    _sc_body,
    out_type=jax.ShapeDtypeStruct((NE, NC, ROWS, WF), jnp.float32),
    mesh=plsc.VectorSubcoreMesh(core_axis_name="c", subcore_axis_name="s"),
    scratch_types=[
        pltpu.VMEM((NBUF, 2, CHUNK), jnp.int32),
        pltpu.VMEM((CHUNK, WF), jnp.float32),
        pltpu.VMEM((CHUNK, WF), jnp.float32),
        pltpu.VMEM((CHUNK, 2 * H), jnp.float32),
        pltpu.VMEM((CHUNK, 2 * H), jnp.float32),
        pltpu.VMEM_SHARED((ROWS, WF), jnp.float32),
        pltpu.SemaphoreType.DMA,
        pltpu.SemaphoreType.DMA,
        pltpu.SemaphoreType.DMA,
        pltpu.SemaphoreType.DMA,
    ],
    compiler_params=pltpu.CompilerParams(use_tc_tiling_on_sc=False,
                                         needs_layout_passes=False),
)


def _combine_body(acc_ref, ha0, ha1, ha2, ad0, ad1, ad2,
                  b0, b1, b2, exp8_ref, attw_ref, attb_ref, out_ref):
    exp8 = exp8_ref[...]
    attw = attw_ref[...]
    accv = acc_ref[...]
    has = (ha0, ha1, ha2)
    adt = (ad0, ad1, ad2)
    bs = (b0, b1, b2)
    res = jnp.zeros((BLK, D), jnp.float32)
    for t in range(NE):
        m = accv[t, 0, :, :D] + accv[t, 1, :, :D]
        den = accv[t, 0, :, D:D + H] + accv[t, 1, :, D:D + H]
        hav = has[t][...]
        h = hav[:, :D]
        a_s = hav[:, D:D + H]
        a_d = adt[t][...][:, :H]
        e = a_s + a_d
        e = jnp.where(e > 0, e, e * jnp.float32(0.2))
        wself = jnp.exp(e)
        den_t = den + wself
        wx = jnp.dot(wself, exp8, preferred_element_type=jnp.float32)
        m_t = m + wx * h
        dfull = jnp.dot(den_t, exp8, preferred_element_type=jnp.float32)
        o = m_t / (dfull + jnp.float32(1e-16)) + bs[t][...]
        att = jnp.sum(o * attw, axis=1, keepdims=True) + attb_ref[...]
        res = res + att * o
    out_ref[...] = res


_combine = pl.pallas_call(
    _combine_body,
    grid=(GRID,),
    in_specs=[pl.BlockSpec((NE, NC, BLK, WF), lambda i: (0, 0, i, 0))]
    + [pl.BlockSpec((BLK, WF), lambda i: (i, 0))] * 3
    + [pl.BlockSpec((BLK, 2 * H), lambda i: (i, 0))] * 3
    + [pl.BlockSpec((1, D), lambda i: (0, 0))] * 3
    + [pl.BlockSpec((H, D), lambda i: (0, 0)),
       pl.BlockSpec((1, D), lambda i: (0, 0)),
       pl.BlockSpec((1, 1), lambda i: (0, 0))],
    out_specs=pl.BlockSpec((BLK, D), lambda i: (i, 0)),
    out_shape=jax.ShapeDtypeStruct((N, D), jnp.float32),
)


def kernel(x, edge_index_0, edge_index_1, edge_index_2,
           W0, att_src0, att_dst0, bias0,
           W1, att_src1, att_dst1, bias1,
           W2, att_src2, att_dst2, bias2,
           att_w, att_b):
    f32 = jnp.float32
    onehot = (jnp.arange(D, dtype=jnp.int32)[:, None] // C
              == jnp.arange(H, dtype=jnp.int32)[None, :]).astype(f32)
    a_srcs = [onehot * a.reshape(D)[:, None]
              for a in (att_src0, att_src1, att_src2)]
    a_dsts = [onehot * a.reshape(D)[:, None]
              for a in (att_dst0, att_dst1, att_dst2)]

    ha0, ha1, ha2, adt0, adt1, adt2 = _prep(x, W0, W1, W2, *a_srcs, *a_dsts)

    pad = EP - E
    ids = []
    for ei in (edge_index_0, edge_index_1, edge_index_2):
        s = jnp.concatenate(
            [ei[0], jnp.zeros((pad,), jnp.int32)]).reshape(NW * CPT, CHUNK)
        d = jnp.concatenate(
            [ei[1], jnp.full((pad,), N, jnp.int32)]).reshape(NW * CPT, CHUNK)
        ids.append(jnp.stack([s, d], axis=1))
    zrows = jnp.zeros((RPT, WF), f32)

    acc = _sc_edge(ha0, ha1, ha2, adt0, adt1, adt2,
                   ids[0], ids[1], ids[2], zrows)

    res = _combine(acc, ha0, ha1, ha2, adt0, adt1, adt2,
                   bias0.reshape(1, D), bias1.reshape(1, D),
                   bias2.reshape(1, D), onehot.T,
                   att_w.reshape(1, D), att_b.reshape(1, 1))
    return res
